# baseline (speedup 1.0000x reference)
import jax
import jax.numpy as jnp
from jax import lax
from jax.experimental import pallas as pl
from jax.experimental.pallas import tpu as pltpu

_C = 80
_STRIDES = (8.0, 16.0, 32.0, 64.0)
_HW = (64, 32, 16, 8)
_ROWS = (2560, 640, 160, 40)
_THR = 0.025
_IOU = 0.6
_K = 1000
_NEG = -1.0e30
_TCAP = 1280


def _lane_iprefix(m):
    x = m
    r = m.shape[0]
    for s in (1, 2, 4, 8, 16, 32, 64):
        x = x + jnp.concatenate(
            [jnp.zeros((r, s), jnp.float32), x[:, :128 - s]], axis=1)
    return x


def _row_eprefix(rc):
    x = rc
    r = rc.shape[0]
    s = 1
    while s < r:
        x = x + jnp.concatenate(
            [jnp.zeros((s, 1), jnp.float32), x[:r - s, :]], axis=0)
        s *= 2
    return x - rc


def _nms_body(sc0, sc1, sc2, sc3, bb0, bb1, bb2, bb3, out_ref,
              vr, x1r_, y1r_, x2r_, y2r_, arear, rankr, pr, slotr,
              mscr0, exgr0, extr0, ofgr0, oftr0,
              mscr1, exgr1, extr1, ofgr1, oftr1,
              mscr2, exgr2, extr2, ofgr2, oftr2,
              mscr3, exgr3, extr3, ofgr3, oftr3,
              gvr0, gir0, tir0, gvr1, gir1, tir1,
              gvr2, gir2, tir2, gvr3, gir3, tir3):
    f32 = jnp.float32
    i32 = jnp.int32
    screfs = (sc0, sc1, sc2, sc3)
    bbs = (bb0, bb1, bb2, bb3)
    mscrs = (mscr0, mscr1, mscr2, mscr3)
    exgrs = (exgr0, exgr1, exgr2, exgr3)
    extrs = (extr0, extr1, extr2, extr3)
    ofgrs = (ofgr0, ofgr1, ofgr2, ofgr3)
    oftrs = (oftr0, oftr1, oftr2, oftr3)
    gvrs = (gvr0, gvr1, gvr2, gvr3)
    girs = (gir0, gir1, gir2, gir3)
    tirs = (tir0, tir1, tir2, tir3)

    lane_i = lax.broadcasted_iota(i32, (1, 128), 1)
    lane = lane_i.astype(f32)

    Vl, X1l, Y1l, X2l, Y2l, LBl = [], [], [], [], [], []
    for l in range(4):
        stride = _STRIDES[l]
        H = _HW[l]
        HW = H * H
        R = _ROWS[l]
        mscr, exgr, extr = mscrs[l], exgrs[l], extrs[l]
        ofgr, oftr = ofgrs[l], oftrs[l]
        gvr, gir, tir = gvrs[l], girs[l], tirs[l]

        x = screfs[l][:, :]
        sc = 1.0 / (1.0 + jnp.exp(-x))
        msc = jnp.where(sc > _THR, sc, -1.0)
        u = lax.bitcast_convert_type(msc, i32)
        ku = jnp.where(u < 0, ~u, u | i32(-2147483648))

        def radix_step(i, pg):
            p, g = pg
            b = 31 - i
            mhigh = lax.shift_left(i32(-1), b)
            one = lax.shift_left(i32(1), b)
            p1 = p | one
            c1 = jnp.sum(jnp.where((ku & mhigh) == p1, 1.0, 0.0))
            cond = (g + c1) >= float(_K)
            return (jnp.where(cond, p1, p), jnp.where(cond, g, g + c1))

        p, g = lax.fori_loop(0, 32, radix_step, (i32(0), f32(0.0)))
        u2 = jnp.where(p < 0, p & i32(2147483647), ~p)
        vstar11 = lax.bitcast_convert_type(
            jnp.full((1, 1), u2, i32), f32)

        gtm = msc > vstar11
        tim = msc == vstar11
        gtf = jnp.where(gtm, 1.0, 0.0)
        tif = jnp.where(tim, 1.0, 0.0)
        ipg = _lane_iprefix(gtf)
        ipt = _lane_iprefix(tif)
        mscr[:, :] = msc
        exgr[:, :] = ipg - gtf
        extr[:, :] = ipt - tif
        ofgr[:, :] = _row_eprefix(ipg[:, 127:128])
        oftr[:, :] = _row_eprefix(ipt[:, 127:128])

        gvr[:, :] = jnp.zeros((9, 128), f32)
        gir[:, :] = jnp.zeros((9, 128), f32)
        tir[:, :] = jnp.zeros((12, 128), f32)

        d3g = (lax.broadcasted_iota(i32, (2, 128, 128), 0) * 128 +
               lax.broadcasted_iota(i32, (2, 128, 128), 1)).astype(f32)

        def scat(w, carry):
            og = jnp.sum(ofgr[pl.ds(w, 1), :])
            ot = jnp.sum(oftr[pl.ds(w, 1), :])
            m = mscr[pl.ds(w, 1), :]
            eg = exgr[pl.ds(w, 1), :]
            et = extr[pl.ds(w, 1), :]
            gm = m > vstar11
            tm = m == vstar11
            cg = jnp.sum(jnp.where(gm, 1.0, 0.0))
            ct = jnp.sum(jnp.where(tm, 1.0, 0.0))
            idxf = lax.convert_element_type(w * 128 + lane_i, f32)

            @pl.when(cg > 0.0)
            def _():
                qg = jnp.floor(og * (1.0 / 128.0))
                qi = qg.astype(i32)
                drel = (og - qg * 128.0) + eg
                hit = jnp.where((d3g == drel[0][None, None, :])
                                & gm[0][None, None, :], 1.0, 0.0)
                v2 = jnp.sum(hit * m[0][None, None, :], axis=2)
                i2 = jnp.sum(hit * idxf[0][None, None, :], axis=2)
                gvr[pl.ds(qi, 2), :] = gvr[pl.ds(qi, 2), :] + v2
                gir[pl.ds(qi, 2), :] = gir[pl.ds(qi, 2), :] + i2

            @pl.when((ct > 0.0) & (ot < float(_TCAP)))
            def _():
                qt = jnp.floor(ot * (1.0 / 128.0))
                qi = qt.astype(i32)
                drel = (ot - qt * 128.0) + et
                hit = jnp.where((d3g == drel[0][None, None, :])
                                & tm[0][None, None, :], 1.0, 0.0)
                i2 = jnp.sum(hit * idxf[0][None, None, :], axis=2)
                tir[pl.ds(qi, 2), :] = tir[pl.ds(qi, 2), :] + i2

            return carry

        lax.fori_loop(0, R, scat, 0)

        gv = gvr[:8, :]
        gi = gir[:8, :]
        ti_buf = tir[:, :]
        tpos = (lax.broadcasted_iota(i32, (12, 128), 0) * 128 +
                lax.broadcasted_iota(i32, (12, 128), 1)).astype(f32)

        tv_rows, ti_rows = [], []
        for r in range(8):
            vi = gv[r][:, None, None]
            ii = gi[r][:, None, None]
            grk = jnp.sum(jnp.sum(jnp.where(
                (gv[None, :, :] > vi)
                | ((gv[None, :, :] == vi) & (gi[None, :, :] < ii)),
                1.0, 0.0), axis=2), axis=1)
            tv_rows.append(grk[None, :])
        grank = jnp.concatenate(tv_rows, axis=0)

        tv_rows, ti_rows = [], []
        for r in range(8):
            dest = (lax.broadcasted_iota(i32, (128, 8, 128), 1) * 0
                    + r * 128
                    + lax.broadcasted_iota(i32, (128, 8, 128), 0)).astype(f32)
            hitg = jnp.where(grank[None, :, :] == dest, 1.0, 0.0)
            tvrow = jnp.sum(jnp.sum(hitg * gv[None, :, :], axis=2), axis=1)
            tirow = jnp.sum(jnp.sum(hitg * gi[None, :, :], axis=2), axis=1)
            dest1 = (r * 128 + lax.broadcasted_iota(
                i32, (128, 12, 128), 0)).astype(f32)
            okt = (dest1 >= g) & (dest1 < float(_K))
            hitt = jnp.where(
                (tpos[None, :, :] == (dest1 - g)) & okt, 1.0, 0.0)
            tirow = tirow + jnp.sum(jnp.sum(hitt * ti_buf[None, :, :],
                                            axis=2), axis=1)
            tvrow = tvrow + jnp.sum(vstar11) * jnp.sum(
                jnp.sum(hitt, axis=2), axis=1)
            tv_rows.append(tvrow[None, :])
            ti_rows.append(tirow[None, :])
        tv = jnp.concatenate(tv_rows, axis=0)
        ti = jnp.concatenate(ti_rows, axis=0)

        bb = bbs[l][:, :]
        dists = []
        for k in range(4):
            gseg = bb[:, 8 * k:8 * k + 8]
            mx = jnp.max(gseg, axis=1, keepdims=True)
            e = jnp.exp(gseg - mx)
            ssum = jnp.sum(e, axis=1)
            w8 = lax.broadcasted_iota(i32, (HW, 8), 1).astype(f32)
            num = jnp.sum(e * w8, axis=1)
            dists.append(num / ssum * stride)

        kq = jnp.floor((ti + 0.5) * (1.0 / _C))
        lb = ti - kq * _C
        rowi = jnp.floor((kq + 0.5) * (1.0 / H))
        coli = kq - rowi * H
        px = (coli + 0.5) * stride
        py = (rowi + 0.5) * stride

        pos_l = (lax.broadcasted_iota(i32, (8, 128), 0) * 128 +
                 lax.broadcasted_iota(i32, (8, 128), 1)).astype(f32)
        ispad = pos_l >= float(_K)
        V = jnp.where(ispad, -jnp.inf, tv)

        x1r, y1r, x2r, y2r = [], [], [], []
        for r in range(8):
            kr = kq[r]
            ih = lax.broadcasted_iota(i32, (HW, 128), 0).astype(f32)
            oh = jnp.where(ih == kr[None, :], 1.0, 0.0)
            d0 = jnp.sum(oh * dists[0][:, None], axis=0)
            d1 = jnp.sum(oh * dists[1][:, None], axis=0)
            d2 = jnp.sum(oh * dists[2][:, None], axis=0)
            d3 = jnp.sum(oh * dists[3][:, None], axis=0)
            x1r.append((px[r] - d0)[None, :])
            y1r.append((py[r] - d1)[None, :])
            x2r.append((px[r] + d2)[None, :])
            y2r.append((py[r] + d3)[None, :])
        X1 = jnp.where(ispad, _NEG, jnp.concatenate(x1r, axis=0))
        Y1 = jnp.where(ispad, _NEG, jnp.concatenate(y1r, axis=0))
        X2 = jnp.where(ispad, _NEG, jnp.concatenate(x2r, axis=0))
        Y2 = jnp.where(ispad, _NEG, jnp.concatenate(y2r, axis=0))
        Vl.append(V); X1l.append(X1); Y1l.append(Y1)
        X2l.append(X2); Y2l.append(Y2); LBl.append(jnp.where(ispad, 0.0, lb))

    V = jnp.concatenate(Vl, axis=0)
    X1 = jnp.concatenate(X1l, axis=0)
    Y1 = jnp.concatenate(Y1l, axis=0)
    X2 = jnp.concatenate(X2l, axis=0)
    Y2 = jnp.concatenate(Y2l, axis=0)
    LB = jnp.concatenate(LBl, axis=0)

    POS = (lax.broadcasted_iota(i32, (32, 128), 0) * 128 +
           lax.broadcasted_iota(i32, (32, 128), 1)).astype(f32)
    VALID = jnp.where(V > _THR, 1.0, 0.0)

    gmax = jnp.max(jnp.maximum(jnp.maximum(X1, X2), jnp.maximum(Y1, Y2)))
    off = LB * (gmax + 1.0)
    sx1 = X1 + off
    sy1 = Y1 + off
    sx2 = X2 + off
    sy2 = Y2 + off
    area = jnp.maximum(sx2 - sx1, 0.0) * jnp.maximum(sy2 - sy1, 0.0)

    vr[:, :] = V
    x1r_[:, :] = sx1
    y1r_[:, :] = sy1
    x2r_[:, :] = sx2
    y2r_[:, :] = sy2
    arear[:, :] = area

    def rbody(w, carry):
        vi = vr[pl.ds(w, 1), :][0][:, None, None]
        pi = (lax.convert_element_type(w, f32) * 128.0 + lane)[0][:, None, None]
        gt = (V[None, :, :] > vi)
        eq = (V[None, :, :] == vi) & (POS[None, :, :] < pi)
        cnt = jnp.sum(jnp.sum(jnp.where(gt | eq, 1.0, 0.0), axis=2), axis=1)
        rankr[pl.ds(w, 1), :] = cnt[None, :]
        return carry

    lax.fori_loop(0, 32, rbody, 0)
    RANK = rankr[:, :]

    sh = lax.broadcasted_iota(i32, (1, 32, 1), 1)

    def pbody(w, carry):
        xi1 = x1r_[pl.ds(w, 1), :][0][:, None, None]
        yi1 = y1r_[pl.ds(w, 1), :][0][:, None, None]
        xi2 = x2r_[pl.ds(w, 1), :][0][:, None, None]
        yi2 = y2r_[pl.ds(w, 1), :][0][:, None, None]
        ai = arear[pl.ds(w, 1), :][0][:, None, None]
        ri = rankr[pl.ds(w, 1), :][0][:, None, None]
        ix1 = jnp.maximum(xi1, sx1[None, :, :])
        iy1 = jnp.maximum(yi1, sy1[None, :, :])
        ix2 = jnp.minimum(xi2, sx2[None, :, :])
        iy2 = jnp.minimum(yi2, sy2[None, :, :])
        inter = jnp.maximum(ix2 - ix1, 0.0) * jnp.maximum(iy2 - iy1, 0.0)
        union = ai + area[None, :, :] - inter
        iou = inter / jnp.maximum(union, 1e-9)
        cond = (iou > _IOU) & (RANK[None, :, :] < ri)
        pblk = jnp.sum(lax.shift_left(cond.astype(i32), sh), axis=1)
        pr[pl.ds(w, 1), :, :] = pblk[None]
        return carry

    lax.fori_loop(0, 32, pbody, 0)
    P = pr[:, :, :]

    shw = lax.broadcasted_iota(i32, (32, 128), 0)

    def wcond(st):
        return st[1]

    def wbody(st):
        keep, _ = st
        kw = jnp.sum(lax.shift_left(keep.astype(i32), shw), axis=0)
        a = P & kw[None, None, :]
        ne = jnp.where(a != 0, 1.0, 0.0)
        sup = jnp.max(ne, axis=2)
        newk = VALID * (1.0 - sup)
        ch = jnp.max(jnp.abs(newk - keep)) > 0.0
        return (newk, ch)

    keep, _ = lax.while_loop(wcond, wbody, (VALID, jnp.bool_(True)))

    def sbody(w, carry):
        ri = rankr[pl.ds(w, 1), :][0][:, None, None]
        lt = jnp.where(RANK[None, :, :] < ri, 1.0, 0.0)
        cnt = jnp.sum(jnp.sum(keep[None, :, :] * lt, axis=2), axis=1)
        slotr[pl.ds(w, 1), :] = cnt[None, :]
        return carry

    lax.fori_loop(0, 32, sbody, 0)
    SLOT = slotr[:, :]
    ktot = jnp.sum(keep)

    sB = lax.broadcasted_iota(i32, (128, 32, 128), 0).astype(f32)
    hit = keep[None, :, :] * jnp.where(SLOT[None, :, :] == sB, 1.0, 0.0)
    fill = jnp.where((sB >= ktot) & (RANK[None, :, :] == 0.0), 1.0, 0.0)
    O = hit + fill

    Vout = jnp.maximum(V, _NEG)
    rows = []
    for ch in (LB, X1, Y1, X2, Y2, Vout):
        rows.append(jnp.sum(jnp.sum(O * ch[None, :, :], axis=2), axis=1)[None, :])
    rows.append(jnp.zeros((2, 128), f32))
    out_ref[:, :] = jnp.concatenate(rows, axis=0)


def kernel(cls_s0, cls_s1, cls_s2, cls_s3, bbox_s0, bbox_s1, bbox_s2, bbox_s3, orig_h, orig_w):
    clss = (cls_s0, cls_s1, cls_s2, cls_s3)
    bbs = (bbox_s0, bbox_s1, bbox_s2, bbox_s3)
    args = []
    for l in range(4):
        hw = _HW[l] * _HW[l]
        args.append(clss[l][0].transpose(1, 2, 0).reshape(_ROWS[l], 128))
    for l in range(4):
        hw = _HW[l] * _HW[l]
        args.append(bbs[l][0].transpose(1, 2, 0).reshape(hw, 32))

    f32 = jnp.float32
    scratch = [
        pltpu.VMEM((32, 128), f32),
        pltpu.VMEM((32, 128), f32),
        pltpu.VMEM((32, 128), f32),
        pltpu.VMEM((32, 128), f32),
        pltpu.VMEM((32, 128), f32),
        pltpu.VMEM((32, 128), f32),
        pltpu.VMEM((32, 128), f32),
        pltpu.VMEM((32, 128, 128), jnp.int32),
        pltpu.VMEM((32, 128), f32),
    ]
    for l in range(4):
        R = _ROWS[l]
        scratch += [
            pltpu.VMEM((R, 128), f32),
            pltpu.VMEM((R, 128), f32),
            pltpu.VMEM((R, 128), f32),
            pltpu.VMEM((R, 1), f32),
            pltpu.VMEM((R, 1), f32),
        ]
    for l in range(4):
        scratch += [
            pltpu.VMEM((9, 128), f32),
            pltpu.VMEM((9, 128), f32),
            pltpu.VMEM((12, 128), f32),
        ]

    out = pl.pallas_call(
        _nms_body,
        out_shape=jax.ShapeDtypeStruct((8, 128), jnp.float32),
        scratch_shapes=scratch,
    )(*args)

    labels = out[0, :100].astype(jnp.int32)
    in_w = float(cls_s0.shape[-1]) * 8.0
    in_h = float(cls_s0.shape[-2]) * 8.0
    scale = jnp.stack([orig_w / in_w, orig_h / in_h,
                       orig_w / in_w, orig_h / in_h]).astype(jnp.float32)
    boxes = jnp.transpose(out[1:5, :100]) * scale[None, :]
    scores = out[5, :100]
    return (labels, boxes, scores)


# speedup vs baseline: 1.3285x; 1.3285x over previous
import jax
import jax.numpy as jnp
from jax import lax
from jax.experimental import pallas as pl
from jax.experimental.pallas import tpu as pltpu

_C = 80
_STRIDES = (8.0, 16.0, 32.0, 64.0)
_HW = (64, 32, 16, 8)
_ROWS = (2560, 640, 160, 40)
_THR = 0.025
_IOU = 0.6
_K = 1000
_NEG = -1.0e30
_TCAP = 1280


def _lane_iprefix(m):
    x = m
    r = m.shape[0]
    for s in (1, 2, 4, 8, 16, 32, 64):
        x = x + jnp.concatenate(
            [jnp.zeros((r, s), jnp.float32), x[:, :128 - s]], axis=1)
    return x


def _row_eprefix(rc):
    x = rc
    r = rc.shape[0]
    s = 1
    while s < r:
        x = x + jnp.concatenate(
            [jnp.zeros((s, 1), jnp.float32), x[:r - s, :]], axis=0)
        s *= 2
    return x - rc


def _nms_body(sc0, sc1, sc2, sc3, bb0, bb1, bb2, bb3, out_ref,
              vr, x1r_, y1r_, x2r_, y2r_, arear, rankr, pr, slotr,
              mscr0, exgr0, extr0, ofgr0, oftr0, cgr0, ctr0,
              mscr1, exgr1, extr1, ofgr1, oftr1, cgr1, ctr1,
              mscr2, exgr2, extr2, ofgr2, oftr2, cgr2, ctr2,
              mscr3, exgr3, extr3, ofgr3, oftr3, cgr3, ctr3,
              gvr0, gir0, tir0, gvr1, gir1, tir1,
              gvr2, gir2, tir2, gvr3, gir3, tir3):
    f32 = jnp.float32
    i32 = jnp.int32
    screfs = (sc0, sc1, sc2, sc3)
    bbs = (bb0, bb1, bb2, bb3)
    mscrs = (mscr0, mscr1, mscr2, mscr3)
    exgrs = (exgr0, exgr1, exgr2, exgr3)
    extrs = (extr0, extr1, extr2, extr3)
    ofgrs = (ofgr0, ofgr1, ofgr2, ofgr3)
    oftrs = (oftr0, oftr1, oftr2, oftr3)
    cgrs = (cgr0, cgr1, cgr2, cgr3)
    ctrs = (ctr0, ctr1, ctr2, ctr3)
    gvrs = (gvr0, gvr1, gvr2, gvr3)
    girs = (gir0, gir1, gir2, gir3)
    tirs = (tir0, tir1, tir2, tir3)

    lane_i = lax.broadcasted_iota(i32, (1, 128), 1)
    lane = lane_i.astype(f32)

    Vl, X1l, Y1l, X2l, Y2l, LBl = [], [], [], [], [], []
    for l in range(4):
        stride = _STRIDES[l]
        H = _HW[l]
        HW = H * H
        R = _ROWS[l]
        mscr, exgr, extr = mscrs[l], exgrs[l], extrs[l]
        ofgr, oftr = ofgrs[l], oftrs[l]
        cgr, ctr = cgrs[l], ctrs[l]
        gvr, gir, tir = gvrs[l], girs[l], tirs[l]

        x = screfs[l][:, :]
        sc = 1.0 / (1.0 + jnp.exp(-x))
        msc = jnp.where(sc > _THR, sc, -1.0)
        u = lax.bitcast_convert_type(msc, i32)
        ku = jnp.where(u < 0, ~u, u | i32(-2147483648))

        def radix_step(i, pg):
            p, g = pg
            b = 31 - i
            mhigh = lax.shift_left(i32(-1), b)
            one = lax.shift_left(i32(1), b)
            p1 = p | one
            c1 = jnp.sum(jnp.where((ku & mhigh) == p1, 1.0, 0.0))
            cond = (g + c1) >= float(_K)
            return (jnp.where(cond, p1, p), jnp.where(cond, g, g + c1))

        p, g = lax.fori_loop(0, 32, radix_step, (i32(0), f32(0.0)))
        u2 = jnp.where(p < 0, p & i32(2147483647), ~p)
        vstar11 = lax.bitcast_convert_type(
            jnp.full((1, 1), u2, i32), f32)

        gtm = msc > vstar11
        tim = msc == vstar11
        gtf = jnp.where(gtm, 1.0, 0.0)
        tif = jnp.where(tim, 1.0, 0.0)
        ipg = _lane_iprefix(gtf)
        ipt = _lane_iprefix(tif)
        mscr[:, :] = msc
        exgr[:, :] = ipg - gtf
        extr[:, :] = ipt - tif
        ofgr[:, :] = _row_eprefix(ipg[:, 127:128])
        oftr[:, :] = _row_eprefix(ipt[:, 127:128])
        cgr[:, :] = ipg[:, 127:128]
        ctr[:, :] = ipt[:, 127:128]

        gvr[:, :] = jnp.zeros((9, 128), f32)
        gir[:, :] = jnp.zeros((9, 128), f32)
        tir[:, :] = jnp.zeros((12, 128), f32)

        d3g = (lax.broadcasted_iota(i32, (2, 128, 128), 0) * 128 +
               lax.broadcasted_iota(i32, (2, 128, 128), 1)).astype(f32)

        def scat(w, carry):
            cg = jnp.sum(cgr[pl.ds(w, 1), :])
            ct = jnp.sum(ctr[pl.ds(w, 1), :])

            @pl.when(cg > 0.0)
            def _():
                og = jnp.sum(ofgr[pl.ds(w, 1), :])
                m = mscr[pl.ds(w, 1), :]
                eg = exgr[pl.ds(w, 1), :]
                gm = m > vstar11
                idxf = lax.convert_element_type(w * 128 + lane_i, f32)
                qg = jnp.floor(og * (1.0 / 128.0))
                qi = qg.astype(i32)
                drel = (og - qg * 128.0) + eg
                hit = jnp.where((d3g == drel[0][None, None, :])
                                & gm[0][None, None, :], 1.0, 0.0)
                v2 = lax.dot_general(hit, m[0], (((2,), (0,)), ((), ())),
                                     preferred_element_type=f32)
                i2 = lax.dot_general(hit, idxf[0], (((2,), (0,)), ((), ())),
                                     preferred_element_type=f32)
                gvr[pl.ds(qi, 2), :] = gvr[pl.ds(qi, 2), :] + v2
                gir[pl.ds(qi, 2), :] = gir[pl.ds(qi, 2), :] + i2

            @pl.when(ct > 0.0)
            def _():
                ot = jnp.sum(oftr[pl.ds(w, 1), :])

                @pl.when(ot < float(_TCAP))
                def _():
                    m = mscr[pl.ds(w, 1), :]
                    et = extr[pl.ds(w, 1), :]
                    tm = m == vstar11
                    idxf = lax.convert_element_type(w * 128 + lane_i, f32)
                    qt = jnp.floor(ot * (1.0 / 128.0))
                    qi = qt.astype(i32)
                    drel = (ot - qt * 128.0) + et
                    hit = jnp.where((d3g == drel[0][None, None, :])
                                    & tm[0][None, None, :], 1.0, 0.0)
                    i2 = lax.dot_general(hit, idxf[0],
                                         (((2,), (0,)), ((), ())),
                                         preferred_element_type=f32)
                    tir[pl.ds(qi, 2), :] = tir[pl.ds(qi, 2), :] + i2

                return None

            return carry

        lax.fori_loop(0, R, scat, 0)

        gv = gvr[:8, :]
        gi = gir[:8, :]
        ti_buf = tir[:, :]
        tpos = (lax.broadcasted_iota(i32, (12, 128), 0) * 128 +
                lax.broadcasted_iota(i32, (12, 128), 1)).astype(f32)

        tv_rows, ti_rows = [], []
        for r in range(8):
            vi = gv[r][:, None, None]
            ii = gi[r][:, None, None]
            grk = jnp.sum(jnp.sum(jnp.where(
                (gv[None, :, :] > vi)
                | ((gv[None, :, :] == vi) & (gi[None, :, :] < ii)),
                1.0, 0.0), axis=2), axis=1)
            tv_rows.append(grk[None, :])
        grank = jnp.concatenate(tv_rows, axis=0)

        tv_rows, ti_rows = [], []
        for r in range(8):
            dest = (lax.broadcasted_iota(i32, (128, 8, 128), 1) * 0
                    + r * 128
                    + lax.broadcasted_iota(i32, (128, 8, 128), 0)).astype(f32)
            hitg = jnp.where(grank[None, :, :] == dest, 1.0, 0.0)
            tvrow = jnp.sum(jnp.sum(hitg * gv[None, :, :], axis=2), axis=1)
            tirow = jnp.sum(jnp.sum(hitg * gi[None, :, :], axis=2), axis=1)
            dest1 = (r * 128 + lax.broadcasted_iota(
                i32, (128, 12, 128), 0)).astype(f32)
            okt = (dest1 >= g) & (dest1 < float(_K))
            hitt = jnp.where(
                (tpos[None, :, :] == (dest1 - g)) & okt, 1.0, 0.0)
            tirow = tirow + jnp.sum(jnp.sum(hitt * ti_buf[None, :, :],
                                            axis=2), axis=1)
            tvrow = tvrow + jnp.sum(vstar11) * jnp.sum(
                jnp.sum(hitt, axis=2), axis=1)
            tv_rows.append(tvrow[None, :])
            ti_rows.append(tirow[None, :])
        tv = jnp.concatenate(tv_rows, axis=0)
        ti = jnp.concatenate(ti_rows, axis=0)

        bb = bbs[l][:, :]
        dists = []
        for k in range(4):
            gseg = bb[:, 8 * k:8 * k + 8]
            mx = jnp.max(gseg, axis=1, keepdims=True)
            e = jnp.exp(gseg - mx)
            ssum = jnp.sum(e, axis=1)
            w8 = lax.broadcasted_iota(i32, (HW, 8), 1).astype(f32)
            num = jnp.sum(e * w8, axis=1)
            dists.append(num / ssum * stride)

        kq = jnp.floor((ti + 0.5) * (1.0 / _C))
        lb = ti - kq * _C
        rowi = jnp.floor((kq + 0.5) * (1.0 / H))
        coli = kq - rowi * H
        px = (coli + 0.5) * stride
        py = (rowi + 0.5) * stride

        pos_l = (lax.broadcasted_iota(i32, (8, 128), 0) * 128 +
                 lax.broadcasted_iota(i32, (8, 128), 1)).astype(f32)
        ispad = pos_l >= float(_K)
        V = jnp.where(ispad, -jnp.inf, tv)

        x1r, y1r, x2r, y2r = [], [], [], []
        for r in range(8):
            kr = kq[r]
            ih = lax.broadcasted_iota(i32, (HW, 128), 0).astype(f32)
            oh = jnp.where(ih == kr[None, :], 1.0, 0.0)
            d0 = jnp.sum(oh * dists[0][:, None], axis=0)
            d1 = jnp.sum(oh * dists[1][:, None], axis=0)
            d2 = jnp.sum(oh * dists[2][:, None], axis=0)
            d3 = jnp.sum(oh * dists[3][:, None], axis=0)
            x1r.append((px[r] - d0)[None, :])
            y1r.append((py[r] - d1)[None, :])
            x2r.append((px[r] + d2)[None, :])
            y2r.append((py[r] + d3)[None, :])
        X1 = jnp.where(ispad, _NEG, jnp.concatenate(x1r, axis=0))
        Y1 = jnp.where(ispad, _NEG, jnp.concatenate(y1r, axis=0))
        X2 = jnp.where(ispad, _NEG, jnp.concatenate(x2r, axis=0))
        Y2 = jnp.where(ispad, _NEG, jnp.concatenate(y2r, axis=0))
        Vl.append(V); X1l.append(X1); Y1l.append(Y1)
        X2l.append(X2); Y2l.append(Y2); LBl.append(jnp.where(ispad, 0.0, lb))

    V = jnp.concatenate(Vl, axis=0)
    X1 = jnp.concatenate(X1l, axis=0)
    Y1 = jnp.concatenate(Y1l, axis=0)
    X2 = jnp.concatenate(X2l, axis=0)
    Y2 = jnp.concatenate(Y2l, axis=0)
    LB = jnp.concatenate(LBl, axis=0)

    POS = (lax.broadcasted_iota(i32, (32, 128), 0) * 128 +
           lax.broadcasted_iota(i32, (32, 128), 1)).astype(f32)
    VALID = jnp.where(V > _THR, 1.0, 0.0)

    gmax = jnp.max(jnp.maximum(jnp.maximum(X1, X2), jnp.maximum(Y1, Y2)))
    off = LB * (gmax + 1.0)
    sx1 = X1 + off
    sy1 = Y1 + off
    sx2 = X2 + off
    sy2 = Y2 + off
    area = jnp.maximum(sx2 - sx1, 0.0) * jnp.maximum(sy2 - sy1, 0.0)

    vr[:, :] = V
    x1r_[:, :] = sx1
    y1r_[:, :] = sy1
    x2r_[:, :] = sx2
    y2r_[:, :] = sy2
    arear[:, :] = area

    def rbody(w, carry):
        vi = vr[pl.ds(w, 1), :][0][:, None, None]
        pi = (lax.convert_element_type(w, f32) * 128.0 + lane)[0][:, None, None]
        gt = (V[None, :, :] > vi)
        eq = (V[None, :, :] == vi) & (POS[None, :, :] < pi)
        cnt = jnp.sum(jnp.sum(jnp.where(gt | eq, 1.0, 0.0), axis=2), axis=1)
        rankr[pl.ds(w, 1), :] = cnt[None, :]
        return carry

    lax.fori_loop(0, 32, rbody, 0)
    RANK = rankr[:, :]

    sh = lax.broadcasted_iota(i32, (1, 32, 1), 1)

    def pbody(w, carry):
        xi1 = x1r_[pl.ds(w, 1), :][0][:, None, None]
        yi1 = y1r_[pl.ds(w, 1), :][0][:, None, None]
        xi2 = x2r_[pl.ds(w, 1), :][0][:, None, None]
        yi2 = y2r_[pl.ds(w, 1), :][0][:, None, None]
        ai = arear[pl.ds(w, 1), :][0][:, None, None]
        ri = rankr[pl.ds(w, 1), :][0][:, None, None]
        ix1 = jnp.maximum(xi1, sx1[None, :, :])
        iy1 = jnp.maximum(yi1, sy1[None, :, :])
        ix2 = jnp.minimum(xi2, sx2[None, :, :])
        iy2 = jnp.minimum(yi2, sy2[None, :, :])
        inter = jnp.maximum(ix2 - ix1, 0.0) * jnp.maximum(iy2 - iy1, 0.0)
        union = ai + area[None, :, :] - inter
        iou = inter / jnp.maximum(union, 1e-9)
        cond = (iou > _IOU) & (RANK[None, :, :] < ri)
        pblk = jnp.sum(lax.shift_left(cond.astype(i32), sh), axis=1)
        pr[pl.ds(w, 1), :, :] = pblk[None]
        return carry

    lax.fori_loop(0, 32, pbody, 0)
    P = pr[:, :, :]

    shw = lax.broadcasted_iota(i32, (32, 128), 0)

    def wcond(st):
        return st[1]

    def wbody(st):
        keep, _ = st
        kw = jnp.sum(lax.shift_left(keep.astype(i32), shw), axis=0)
        a = P & kw[None, None, :]
        ne = jnp.where(a != 0, 1.0, 0.0)
        sup = jnp.max(ne, axis=2)
        newk = VALID * (1.0 - sup)
        ch = jnp.max(jnp.abs(newk - keep)) > 0.0
        return (newk, ch)

    keep, _ = lax.while_loop(wcond, wbody, (VALID, jnp.bool_(True)))

    def sbody(w, carry):
        ri = rankr[pl.ds(w, 1), :][0][:, None, None]
        lt = jnp.where(RANK[None, :, :] < ri, 1.0, 0.0)
        cnt = jnp.sum(jnp.sum(keep[None, :, :] * lt, axis=2), axis=1)
        slotr[pl.ds(w, 1), :] = cnt[None, :]
        return carry

    lax.fori_loop(0, 32, sbody, 0)
    SLOT = slotr[:, :]
    ktot = jnp.sum(keep)

    sB = lax.broadcasted_iota(i32, (128, 32, 128), 0).astype(f32)
    hit = keep[None, :, :] * jnp.where(SLOT[None, :, :] == sB, 1.0, 0.0)
    fill = jnp.where((sB >= ktot) & (RANK[None, :, :] == 0.0), 1.0, 0.0)
    O = hit + fill

    Vout = jnp.maximum(V, _NEG)
    rows = []
    for ch in (LB, X1, Y1, X2, Y2, Vout):
        rows.append(jnp.sum(jnp.sum(O * ch[None, :, :], axis=2), axis=1)[None, :])
    rows.append(jnp.zeros((2, 128), f32))
    out_ref[:, :] = jnp.concatenate(rows, axis=0)


def kernel(cls_s0, cls_s1, cls_s2, cls_s3, bbox_s0, bbox_s1, bbox_s2, bbox_s3, orig_h, orig_w):
    clss = (cls_s0, cls_s1, cls_s2, cls_s3)
    bbs = (bbox_s0, bbox_s1, bbox_s2, bbox_s3)
    args = []
    for l in range(4):
        hw = _HW[l] * _HW[l]
        args.append(clss[l][0].transpose(1, 2, 0).reshape(_ROWS[l], 128))
    for l in range(4):
        hw = _HW[l] * _HW[l]
        args.append(bbs[l][0].transpose(1, 2, 0).reshape(hw, 32))

    f32 = jnp.float32
    scratch = [
        pltpu.VMEM((32, 128), f32),
        pltpu.VMEM((32, 128), f32),
        pltpu.VMEM((32, 128), f32),
        pltpu.VMEM((32, 128), f32),
        pltpu.VMEM((32, 128), f32),
        pltpu.VMEM((32, 128), f32),
        pltpu.VMEM((32, 128), f32),
        pltpu.VMEM((32, 128, 128), jnp.int32),
        pltpu.VMEM((32, 128), f32),
    ]
    for l in range(4):
        R = _ROWS[l]
        scratch += [
            pltpu.VMEM((R, 128), f32),
            pltpu.VMEM((R, 128), f32),
            pltpu.VMEM((R, 128), f32),
            pltpu.VMEM((R, 1), f32),
            pltpu.VMEM((R, 1), f32),
            pltpu.VMEM((R, 1), f32),
            pltpu.VMEM((R, 1), f32),
        ]
    for l in range(4):
        scratch += [
            pltpu.VMEM((9, 128), f32),
            pltpu.VMEM((9, 128), f32),
            pltpu.VMEM((12, 128), f32),
        ]

    out = pl.pallas_call(
        _nms_body,
        out_shape=jax.ShapeDtypeStruct((8, 128), jnp.float32),
        scratch_shapes=scratch,
    )(*args)

    labels = out[0, :100].astype(jnp.int32)
    in_w = float(cls_s0.shape[-1]) * 8.0
    in_h = float(cls_s0.shape[-2]) * 8.0
    scale = jnp.stack([orig_w / in_w, orig_h / in_h,
                       orig_w / in_w, orig_h / in_h]).astype(jnp.float32)
    boxes = jnp.transpose(out[1:5, :100]) * scale[None, :]
    scores = out[5, :100]
    return (labels, boxes, scores)


# speedup vs baseline: 1.3440x; 1.0116x over previous
import jax
import jax.numpy as jnp
from jax import lax
from jax.experimental import pallas as pl
from jax.experimental.pallas import tpu as pltpu

_C = 80
_STRIDES = (8.0, 16.0, 32.0, 64.0)
_HW = (64, 32, 16, 8)
_ROWS = (2560, 640, 160, 40)
_THR = 0.025
_IOU = 0.6
_K = 1000
_NEG = -1.0e30
_TCAP = 1280


def _lane_iprefix(m):
    x = m
    r = m.shape[0]
    for s in (1, 2, 4, 8, 16, 32, 64):
        x = x + jnp.concatenate(
            [jnp.zeros((r, s), jnp.float32), x[:, :128 - s]], axis=1)
    return x


def _row_eprefix(rc):
    x = rc
    r = rc.shape[0]
    s = 1
    while s < r:
        x = x + jnp.concatenate(
            [jnp.zeros((s, 1), jnp.float32), x[:r - s, :]], axis=0)
        s *= 2
    return x - rc


def _nms_body(sc0, sc1, sc2, sc3, bb0, bb1, bb2, bb3, out_ref,
              vr, x1r_, y1r_, x2r_, y2r_, arear, rankr, pr, slotr,
              mscr0, exgr0, extr0, ofgr0, oftr0, cgr0, ctr0,
              mscr1, exgr1, extr1, ofgr1, oftr1, cgr1, ctr1,
              mscr2, exgr2, extr2, ofgr2, oftr2, cgr2, ctr2,
              mscr3, exgr3, extr3, ofgr3, oftr3, cgr3, ctr3,
              gvr0, gir0, tir0, gvr1, gir1, tir1,
              gvr2, gir2, tir2, gvr3, gir3, tir3):
    f32 = jnp.float32
    i32 = jnp.int32
    screfs = (sc0, sc1, sc2, sc3)
    bbs = (bb0, bb1, bb2, bb3)
    mscrs = (mscr0, mscr1, mscr2, mscr3)
    exgrs = (exgr0, exgr1, exgr2, exgr3)
    extrs = (extr0, extr1, extr2, extr3)
    ofgrs = (ofgr0, ofgr1, ofgr2, ofgr3)
    oftrs = (oftr0, oftr1, oftr2, oftr3)
    cgrs = (cgr0, cgr1, cgr2, cgr3)
    ctrs = (ctr0, ctr1, ctr2, ctr3)
    gvrs = (gvr0, gvr1, gvr2, gvr3)
    girs = (gir0, gir1, gir2, gir3)
    tirs = (tir0, tir1, tir2, tir3)

    lane_i = lax.broadcasted_iota(i32, (1, 128), 1)
    lane = lane_i.astype(f32)

    mscs, kus = [], []
    for l in range(4):
        x = screfs[l][:, :]
        sc = 1.0 / (1.0 + jnp.exp(-x))
        msc = jnp.where(sc > _THR, sc, -1.0)
        u = lax.bitcast_convert_type(msc, i32)
        mscs.append(msc)
        kus.append(jnp.where(u < 0, ~u, u | i32(-2147483648)))

    def radix_step(i, st):
        b = 31 - i
        mhigh = lax.shift_left(i32(-1), b)
        one = lax.shift_left(i32(1), b)
        out = []
        for l in range(4):
            p, g = st[l], st[4 + l]
            p1 = p | one
            c1 = jnp.sum(jnp.where((kus[l] & mhigh) == p1, 1.0, 0.0))
            cond = (g + c1) >= float(_K)
            out.append((jnp.where(cond, p1, p), jnp.where(cond, g, g + c1)))
        return tuple(pg[0] for pg in out) + tuple(pg[1] for pg in out)

    rsel = lax.fori_loop(0, 32, radix_step,
                         (i32(0), i32(0), i32(0), i32(0),
                          f32(0.0), f32(0.0), f32(0.0), f32(0.0)))

    Vl, X1l, Y1l, X2l, Y2l, LBl = [], [], [], [], [], []
    for l in range(4):
        stride = _STRIDES[l]
        H = _HW[l]
        HW = H * H
        R = _ROWS[l]
        mscr, exgr, extr = mscrs[l], exgrs[l], extrs[l]
        ofgr, oftr = ofgrs[l], oftrs[l]
        cgr, ctr = cgrs[l], ctrs[l]
        gvr, gir, tir = gvrs[l], girs[l], tirs[l]
        msc = mscs[l]
        p, g = rsel[l], rsel[4 + l]
        u2 = jnp.where(p < 0, p & i32(2147483647), ~p)
        vstar11 = lax.bitcast_convert_type(
            jnp.full((1, 1), u2, i32), f32)

        gtm = msc > vstar11
        tim = msc == vstar11
        gtf = jnp.where(gtm, 1.0, 0.0)
        tif = jnp.where(tim, 1.0, 0.0)
        ipg = _lane_iprefix(gtf)
        ipt = _lane_iprefix(tif)
        mscr[:, :] = msc
        exgr[:, :] = ipg - gtf
        extr[:, :] = ipt - tif
        ofgr[:, :] = _row_eprefix(ipg[:, 127:128])
        oftr[:, :] = _row_eprefix(ipt[:, 127:128])
        cgr[:, :] = ipg[:, 127:128]
        ctr[:, :] = ipt[:, 127:128]

        gvr[:, :] = jnp.zeros((9, 128), f32)
        gir[:, :] = jnp.zeros((9, 128), f32)
        tir[:, :] = jnp.zeros((12, 128), f32)

        d3g = (lax.broadcasted_iota(i32, (2, 128, 128), 0) * 128 +
               lax.broadcasted_iota(i32, (2, 128, 128), 1)).astype(f32)

        def scat(w, carry):
            cg = jnp.sum(cgr[pl.ds(w, 1), :])
            ct = jnp.sum(ctr[pl.ds(w, 1), :])

            @pl.when(cg > 0.0)
            def _():
                og = jnp.sum(ofgr[pl.ds(w, 1), :])
                m = mscr[pl.ds(w, 1), :]
                eg = exgr[pl.ds(w, 1), :]
                gm = m > vstar11
                idxf = lax.convert_element_type(w * 128 + lane_i, f32)
                qg = jnp.floor(og * (1.0 / 128.0))
                qi = qg.astype(i32)
                drel = (og - qg * 128.0) + eg
                hit = jnp.where((d3g == drel[0][None, None, :])
                                & gm[0][None, None, :], 1.0, 0.0)
                v2 = lax.dot_general(hit, m[0], (((2,), (0,)), ((), ())),
                                     preferred_element_type=f32)
                i2 = lax.dot_general(hit, idxf[0], (((2,), (0,)), ((), ())),
                                     preferred_element_type=f32)
                gvr[pl.ds(qi, 2), :] = gvr[pl.ds(qi, 2), :] + v2
                gir[pl.ds(qi, 2), :] = gir[pl.ds(qi, 2), :] + i2

            @pl.when(ct > 0.0)
            def _():
                ot = jnp.sum(oftr[pl.ds(w, 1), :])

                @pl.when(ot < float(_TCAP))
                def _():
                    m = mscr[pl.ds(w, 1), :]
                    et = extr[pl.ds(w, 1), :]
                    tm = m == vstar11
                    idxf = lax.convert_element_type(w * 128 + lane_i, f32)
                    qt = jnp.floor(ot * (1.0 / 128.0))
                    qi = qt.astype(i32)
                    drel = (ot - qt * 128.0) + et
                    hit = jnp.where((d3g == drel[0][None, None, :])
                                    & tm[0][None, None, :], 1.0, 0.0)
                    i2 = lax.dot_general(hit, idxf[0],
                                         (((2,), (0,)), ((), ())),
                                         preferred_element_type=f32)
                    tir[pl.ds(qi, 2), :] = tir[pl.ds(qi, 2), :] + i2

                return None

            return carry

        lax.fori_loop(0, R, scat, 0)

        gv = gvr[:8, :]
        gi = gir[:8, :]
        ti_buf = tir[:, :]
        tpos = (lax.broadcasted_iota(i32, (12, 128), 0) * 128 +
                lax.broadcasted_iota(i32, (12, 128), 1)).astype(f32)

        tv_rows, ti_rows = [], []
        for r in range(8):
            vi = gv[r][:, None, None]
            ii = gi[r][:, None, None]
            grk = jnp.sum(jnp.sum(jnp.where(
                (gv[None, :, :] > vi)
                | ((gv[None, :, :] == vi) & (gi[None, :, :] < ii)),
                1.0, 0.0), axis=2), axis=1)
            tv_rows.append(grk[None, :])
        grank = jnp.concatenate(tv_rows, axis=0)

        tv_rows, ti_rows = [], []
        for r in range(8):
            dest = (lax.broadcasted_iota(i32, (128, 8, 128), 1) * 0
                    + r * 128
                    + lax.broadcasted_iota(i32, (128, 8, 128), 0)).astype(f32)
            hitg = jnp.where(grank[None, :, :] == dest, 1.0, 0.0)
            tvrow = jnp.sum(jnp.sum(hitg * gv[None, :, :], axis=2), axis=1)
            tirow = jnp.sum(jnp.sum(hitg * gi[None, :, :], axis=2), axis=1)
            dest1 = (r * 128 + lax.broadcasted_iota(
                i32, (128, 12, 128), 0)).astype(f32)
            okt = (dest1 >= g) & (dest1 < float(_K))
            hitt = jnp.where(
                (tpos[None, :, :] == (dest1 - g)) & okt, 1.0, 0.0)
            tirow = tirow + jnp.sum(jnp.sum(hitt * ti_buf[None, :, :],
                                            axis=2), axis=1)
            tvrow = tvrow + jnp.sum(vstar11) * jnp.sum(
                jnp.sum(hitt, axis=2), axis=1)
            tv_rows.append(tvrow[None, :])
            ti_rows.append(tirow[None, :])
        tv = jnp.concatenate(tv_rows, axis=0)
        ti = jnp.concatenate(ti_rows, axis=0)

        bb = bbs[l][:, :]
        dists = []
        for k in range(4):
            gseg = bb[:, 8 * k:8 * k + 8]
            mx = jnp.max(gseg, axis=1, keepdims=True)
            e = jnp.exp(gseg - mx)
            ssum = jnp.sum(e, axis=1)
            w8 = lax.broadcasted_iota(i32, (HW, 8), 1).astype(f32)
            num = jnp.sum(e * w8, axis=1)
            dists.append(num / ssum * stride)

        kq = jnp.floor((ti + 0.5) * (1.0 / _C))
        lb = ti - kq * _C
        rowi = jnp.floor((kq + 0.5) * (1.0 / H))
        coli = kq - rowi * H
        px = (coli + 0.5) * stride
        py = (rowi + 0.5) * stride

        pos_l = (lax.broadcasted_iota(i32, (8, 128), 0) * 128 +
                 lax.broadcasted_iota(i32, (8, 128), 1)).astype(f32)
        ispad = pos_l >= float(_K)
        V = jnp.where(ispad, -jnp.inf, tv)

        x1r, y1r, x2r, y2r = [], [], [], []
        for r in range(8):
            kr = kq[r]
            ih = lax.broadcasted_iota(i32, (HW, 128), 0).astype(f32)
            oh = jnp.where(ih == kr[None, :], 1.0, 0.0)
            d0 = jnp.sum(oh * dists[0][:, None], axis=0)
            d1 = jnp.sum(oh * dists[1][:, None], axis=0)
            d2 = jnp.sum(oh * dists[2][:, None], axis=0)
            d3 = jnp.sum(oh * dists[3][:, None], axis=0)
            x1r.append((px[r] - d0)[None, :])
            y1r.append((py[r] - d1)[None, :])
            x2r.append((px[r] + d2)[None, :])
            y2r.append((py[r] + d3)[None, :])
        X1 = jnp.where(ispad, _NEG, jnp.concatenate(x1r, axis=0))
        Y1 = jnp.where(ispad, _NEG, jnp.concatenate(y1r, axis=0))
        X2 = jnp.where(ispad, _NEG, jnp.concatenate(x2r, axis=0))
        Y2 = jnp.where(ispad, _NEG, jnp.concatenate(y2r, axis=0))
        Vl.append(V); X1l.append(X1); Y1l.append(Y1)
        X2l.append(X2); Y2l.append(Y2); LBl.append(jnp.where(ispad, 0.0, lb))

    V = jnp.concatenate(Vl, axis=0)
    X1 = jnp.concatenate(X1l, axis=0)
    Y1 = jnp.concatenate(Y1l, axis=0)
    X2 = jnp.concatenate(X2l, axis=0)
    Y2 = jnp.concatenate(Y2l, axis=0)
    LB = jnp.concatenate(LBl, axis=0)

    POS = (lax.broadcasted_iota(i32, (32, 128), 0) * 128 +
           lax.broadcasted_iota(i32, (32, 128), 1)).astype(f32)
    VALID = jnp.where(V > _THR, 1.0, 0.0)

    gmax = jnp.max(jnp.maximum(jnp.maximum(X1, X2), jnp.maximum(Y1, Y2)))
    off = LB * (gmax + 1.0)
    sx1 = X1 + off
    sy1 = Y1 + off
    sx2 = X2 + off
    sy2 = Y2 + off
    area = jnp.maximum(sx2 - sx1, 0.0) * jnp.maximum(sy2 - sy1, 0.0)

    vr[:, :] = V
    x1r_[:, :] = sx1
    y1r_[:, :] = sy1
    x2r_[:, :] = sx2
    y2r_[:, :] = sy2
    arear[:, :] = area

    def rbody(w, carry):
        vi = vr[pl.ds(w, 1), :][0][:, None, None]
        pi = (lax.convert_element_type(w, f32) * 128.0 + lane)[0][:, None, None]
        gt = (V[None, :, :] > vi)
        eq = (V[None, :, :] == vi) & (POS[None, :, :] < pi)
        cnt = jnp.sum(jnp.sum(jnp.where(gt | eq, 1.0, 0.0), axis=2), axis=1)
        rankr[pl.ds(w, 1), :] = cnt[None, :]
        return carry

    lax.fori_loop(0, 32, rbody, 0)
    RANK = rankr[:, :]

    sh = lax.broadcasted_iota(i32, (1, 32, 1), 1)

    def pbody(w, carry):
        xi1 = x1r_[pl.ds(w, 1), :][0][:, None, None]
        yi1 = y1r_[pl.ds(w, 1), :][0][:, None, None]
        xi2 = x2r_[pl.ds(w, 1), :][0][:, None, None]
        yi2 = y2r_[pl.ds(w, 1), :][0][:, None, None]
        ai = arear[pl.ds(w, 1), :][0][:, None, None]
        ri = rankr[pl.ds(w, 1), :][0][:, None, None]
        ix1 = jnp.maximum(xi1, sx1[None, :, :])
        iy1 = jnp.maximum(yi1, sy1[None, :, :])
        ix2 = jnp.minimum(xi2, sx2[None, :, :])
        iy2 = jnp.minimum(yi2, sy2[None, :, :])
        inter = jnp.maximum(ix2 - ix1, 0.0) * jnp.maximum(iy2 - iy1, 0.0)
        union = ai + area[None, :, :] - inter
        iou = inter / jnp.maximum(union, 1e-9)
        cond = (iou > _IOU) & (RANK[None, :, :] < ri)
        pblk = jnp.sum(lax.shift_left(cond.astype(i32), sh), axis=1)
        pr[pl.ds(w, 1), :, :] = pblk[None]
        return carry

    lax.fori_loop(0, 32, pbody, 0)
    P = pr[:, :, :]

    shw = lax.broadcasted_iota(i32, (32, 128), 0)

    def wcond(st):
        return st[1]

    def wbody(st):
        keep, _ = st
        kw = jnp.sum(lax.shift_left(keep.astype(i32), shw), axis=0)
        a = P & kw[None, None, :]
        ne = jnp.where(a != 0, 1.0, 0.0)
        sup = jnp.max(ne, axis=2)
        newk = VALID * (1.0 - sup)
        ch = jnp.max(jnp.abs(newk - keep)) > 0.0
        return (newk, ch)

    keep, _ = lax.while_loop(wcond, wbody, (VALID, jnp.bool_(True)))

    def sbody(w, carry):
        ri = rankr[pl.ds(w, 1), :][0][:, None, None]
        lt = jnp.where(RANK[None, :, :] < ri, 1.0, 0.0)
        cnt = jnp.sum(jnp.sum(keep[None, :, :] * lt, axis=2), axis=1)
        slotr[pl.ds(w, 1), :] = cnt[None, :]
        return carry

    lax.fori_loop(0, 32, sbody, 0)
    SLOT = slotr[:, :]
    ktot = jnp.sum(keep)

    sB = lax.broadcasted_iota(i32, (128, 32, 128), 0).astype(f32)
    hit = keep[None, :, :] * jnp.where(SLOT[None, :, :] == sB, 1.0, 0.0)
    fill = jnp.where((sB >= ktot) & (RANK[None, :, :] == 0.0), 1.0, 0.0)
    O = hit + fill

    Vout = jnp.maximum(V, _NEG)
    rows = []
    for ch in (LB, X1, Y1, X2, Y2, Vout):
        rows.append(jnp.sum(jnp.sum(O * ch[None, :, :], axis=2), axis=1)[None, :])
    rows.append(jnp.zeros((2, 128), f32))
    out_ref[:, :] = jnp.concatenate(rows, axis=0)


def kernel(cls_s0, cls_s1, cls_s2, cls_s3, bbox_s0, bbox_s1, bbox_s2, bbox_s3, orig_h, orig_w):
    clss = (cls_s0, cls_s1, cls_s2, cls_s3)
    bbs = (bbox_s0, bbox_s1, bbox_s2, bbox_s3)
    args = []
    for l in range(4):
        hw = _HW[l] * _HW[l]
        args.append(clss[l][0].transpose(1, 2, 0).reshape(_ROWS[l], 128))
    for l in range(4):
        hw = _HW[l] * _HW[l]
        args.append(bbs[l][0].transpose(1, 2, 0).reshape(hw, 32))

    f32 = jnp.float32
    scratch = [
        pltpu.VMEM((32, 128), f32),
        pltpu.VMEM((32, 128), f32),
        pltpu.VMEM((32, 128), f32),
        pltpu.VMEM((32, 128), f32),
        pltpu.VMEM((32, 128), f32),
        pltpu.VMEM((32, 128), f32),
        pltpu.VMEM((32, 128), f32),
        pltpu.VMEM((32, 128, 128), jnp.int32),
        pltpu.VMEM((32, 128), f32),
    ]
    for l in range(4):
        R = _ROWS[l]
        scratch += [
            pltpu.VMEM((R, 128), f32),
            pltpu.VMEM((R, 128), f32),
            pltpu.VMEM((R, 128), f32),
            pltpu.VMEM((R, 1), f32),
            pltpu.VMEM((R, 1), f32),
            pltpu.VMEM((R, 1), f32),
            pltpu.VMEM((R, 1), f32),
        ]
    for l in range(4):
        scratch += [
            pltpu.VMEM((9, 128), f32),
            pltpu.VMEM((9, 128), f32),
            pltpu.VMEM((12, 128), f32),
        ]

    out = pl.pallas_call(
        _nms_body,
        out_shape=jax.ShapeDtypeStruct((8, 128), jnp.float32),
        scratch_shapes=scratch,
    )(*args)

    labels = out[0, :100].astype(jnp.int32)
    in_w = float(cls_s0.shape[-1]) * 8.0
    in_h = float(cls_s0.shape[-2]) * 8.0
    scale = jnp.stack([orig_w / in_w, orig_h / in_h,
                       orig_w / in_w, orig_h / in_h]).astype(jnp.float32)
    boxes = jnp.transpose(out[1:5, :100]) * scale[None, :]
    scores = out[5, :100]
    return (labels, boxes, scores)
